# Initial kernel scaffold; baseline (speedup 1.0000x reference)
import jax
import jax.numpy as jnp
from jax.experimental import pallas as pl


def kernel(ft_user, ft_item, edge_index1, cnt1, edge_index2, Wq1, bq1, Wk1, Wv1, attn_e1, cnt_table1, Wq2, bq2, Wk2, Wv2, attn_e2, Wagg, bagg, Wself):
    raise NotImplementedError("write your pallas kernel here")


# speedup vs baseline: 8.4629x; 8.4629x over previous
import functools

import jax
import jax.numpy as jnp
from jax import lax
from jax.experimental import pallas as pl
from jax.experimental.pallas import tpu as pltpu
from jax.experimental.pallas import tpu_sc as plsc

NC = 2
NS = 16
NW = NC * NS
C = 128
H = 8
HD = 16
D = 128
NP = 10112
NPD = 640


def _mesh():
    return plsc.VectorSubcoreMesh(core_axis_name="c", subcore_axis_name="s")


def _p1_body(E1, E2, N,
             q1t, k1t, ct1, q2t, k2t, attn, src1, dst1, cnt1, src2, dst2, z128,
             ex_out, den_out,
             sidx, didx, d16, cidx, qrows, krows, crows, exbuf, denrow, attnv,
             den_scr, sem0, sem1, sem2):
    c = lax.axis_index("c")
    s = lax.axis_index("s")
    wid = s * NC + c
    RD = NPD // NS
    pltpu.sync_copy(z128.at[pl.ds(s * RD, RD)], den_scr.at[pl.ds(s * RD, RD)])
    pltpu.sync_copy(z128.at[pl.ds(0, C)], denrow)
    pltpu.sync_copy(attn, attnv)
    plsc.subcore_barrier()

    def make_chunk_body(qt, kt, srcv, dstv, use_cnt, et, ebase):
        def chunk_body(i, carry):
            base = (wid + i * NW) * C
            pltpu.sync_copy(srcv.at[pl.ds(base, C)], sidx)
            pltpu.sync_copy(dstv.at[pl.ds(base, C)], didx)
            cp0 = pltpu.async_copy(qt.at[sidx], qrows, sem0)
            cp1 = pltpu.async_copy(kt.at[didx], krows, sem1)
            if use_cnt:
                pltpu.sync_copy(cnt1.at[pl.ds(base, C)], cidx)
                cp2 = pltpu.async_copy(ct1.at[cidx], crows, sem2)
            cp0.wait()
            cp1.wait()
            if use_cnt:
                cp2.wait()

            def block_body(b, bc):
                rows = lax.iota(jnp.int32, 16) + b * 16
                for h in range(H):
                    acc = jnp.zeros((16,), jnp.float32)
                    for t in range(HD):
                        d = h * HD + t
                        dcol = jnp.full((16,), d, jnp.int32)
                        x = (plsc.load_gather(qrows, [rows, dcol])
                             + plsc.load_gather(krows, [rows, dcol]))
                        if use_cnt:
                            x = x + plsc.load_gather(crows, [rows, dcol])
                        sg = 1.0 / (1.0 + jnp.exp(-x))
                        acc = acc + attnv[et, d, :] * sg
                    exh = jnp.exp(acc)
                    plsc.store_scatter(
                        exbuf, [rows, jnp.full((16,), h, jnp.int32)], exh)
                return bc

            lax.fori_loop(0, C // 16, block_body, 0)

            def den_write(b, bc):
                rows = lax.iota(jnp.int32, 16) + b * 16
                dv = plsc.load_gather(didx, [rows])
                plsc.store_scatter(d16, [rows],
                                   lax.shift_right_logical(dv, 4))
                colbase = jnp.bitwise_and(dv, 15) * 8
                for h in range(H):
                    exv = plsc.load_gather(
                        exbuf, [rows, jnp.full((16,), h, jnp.int32)])
                    plsc.store_scatter(denrow, [rows, colbase + h], exv)
                return bc

            lax.fori_loop(0, C // 16, den_write, 0)
            pltpu.sync_copy(exbuf, ex_out.at[pl.ds(ebase + base, C)])
            pltpu.sync_copy(denrow, den_scr.at[d16], add=True)

            def den_clear(b, bc):
                rows = lax.iota(jnp.int32, 16) + b * 16
                dv = plsc.load_gather(didx, [rows])
                colbase = jnp.bitwise_and(dv, 15) * 8
                zv = jnp.zeros((16,), jnp.float32)
                for h in range(H):
                    plsc.store_scatter(denrow, [rows, colbase + h], zv)
                return bc

            lax.fori_loop(0, C // 16, den_clear, 0)
            return carry
        return chunk_body

    n1 = (E1 // C - 1 - wid) // NW + 1
    lax.fori_loop(0, n1, make_chunk_body(q1t, k1t, src1, dst1, True, 0, 0), 0)
    n2 = (E2 // C - 1 - wid) // NW + 1
    lax.fori_loop(0, n2, make_chunk_body(q2t, k2t, src2, dst2, False, 1, E1), 0)
    plsc.subcore_barrier()
    pltpu.sync_copy(den_scr.at[pl.ds(s * RD, RD)],
                    den_out.at[pl.ds(c * NPD + s * RD, RD)])


def _pass1(q1t, k1t, ct1, q2t, k2t, attn, src1, dst1, cnt1, src2, dst2, z128):
    E1 = src1.shape[0]
    E2 = src2.shape[0]
    N = q1t.shape[0]
    f = pl.kernel(
        functools.partial(_p1_body, E1, E2, N),
        out_type=(
            jax.ShapeDtypeStruct((E1 + E2, H), jnp.float32),
            jax.ShapeDtypeStruct((NC * NPD, D), jnp.float32),
        ),
        mesh=_mesh(),
        compiler_params=pltpu.CompilerParams(needs_layout_passes=False),
        scratch_types=[
            pltpu.VMEM((C,), jnp.int32),
            pltpu.VMEM((C,), jnp.int32),
            pltpu.VMEM((C,), jnp.int32),
            pltpu.VMEM((C,), jnp.int32),
            pltpu.VMEM((C, D), jnp.float32),
            pltpu.VMEM((C, D), jnp.float32),
            pltpu.VMEM((C, D), jnp.float32),
            pltpu.VMEM((C, H), jnp.float32),
            pltpu.VMEM((C, D), jnp.float32),
            pltpu.VMEM((2, D, 16), jnp.float32),
            pltpu.VMEM_SHARED((NPD, D), jnp.float32),
            pltpu.SemaphoreType.DMA,
            pltpu.SemaphoreType.DMA,
            pltpu.SemaphoreType.DMA,
        ],
        name="hetero_attn_pass1",
    )
    return f(q1t, k1t, ct1, q2t, k2t, attn, src1, dst1, cnt1, src2, dst2, z128)


def _p2_body(E1, E2, N,
             v1t, v2t, src1, dst1, src2, dst2, exf, z128,
             agg_out,
             sidx, didx, vrows, orows, exb,
             agg_scr, sem0):
    c = lax.axis_index("c")
    s = lax.axis_index("s")
    wid = s * NC + c
    R = NP // NS
    pltpu.sync_copy(z128.at[pl.ds(s * R, R)], agg_scr.at[pl.ds(s * R, R)])
    plsc.subcore_barrier()

    def make_chunk_body(vt, srcv, dstv, ebase):
        def chunk_body(i, carry):
            base = (wid + i * NW) * C
            pltpu.sync_copy(srcv.at[pl.ds(base, C)], sidx)
            pltpu.sync_copy(dstv.at[pl.ds(base, C)], didx)
            cp0 = pltpu.async_copy(vt.at[sidx], vrows, sem0)
            pltpu.sync_copy(exf.at[pl.ds(ebase + base, C)], exb)
            cp0.wait()

            def block_body(b, bc):
                rows = lax.iota(jnp.int32, 16) + b * 16
                for h in range(H):
                    exv = plsc.load_gather(
                        exb, [rows, jnp.full((16,), h, jnp.int32)])
                    for t in range(HD):
                        d = h * HD + t
                        dcol = jnp.full((16,), d, jnp.int32)
                        vv = plsc.load_gather(vrows, [rows, dcol])
                        plsc.store_scatter(orows, [rows, dcol], vv * exv)
                return bc

            lax.fori_loop(0, C // 16, block_body, 0)
            pltpu.sync_copy(orows, agg_scr.at[didx], add=True)
            return carry
        return chunk_body

    n1 = (E1 // C - 1 - wid) // NW + 1
    lax.fori_loop(0, n1, make_chunk_body(v1t, src1, dst1, 0), 0)
    n2 = (E2 // C - 1 - wid) // NW + 1
    lax.fori_loop(0, n2, make_chunk_body(v2t, src2, dst2, E1), 0)
    plsc.subcore_barrier()
    pltpu.sync_copy(agg_scr.at[pl.ds(s * R, R)],
                    agg_out.at[pl.ds(c * NP + s * R, R)])


def _pass2(v1t, v2t, src1, dst1, src2, dst2, exf, z128):
    E1 = src1.shape[0]
    E2 = src2.shape[0]
    N = v1t.shape[0]
    f = pl.kernel(
        functools.partial(_p2_body, E1, E2, N),
        out_type=jax.ShapeDtypeStruct((NC * NP, D), jnp.float32),
        mesh=_mesh(),
        compiler_params=pltpu.CompilerParams(needs_layout_passes=False),
        scratch_types=[
            pltpu.VMEM((C,), jnp.int32),
            pltpu.VMEM((C,), jnp.int32),
            pltpu.VMEM((C, D), jnp.float32),
            pltpu.VMEM((C, D), jnp.float32),
            pltpu.VMEM((C, H), jnp.float32),
            pltpu.VMEM_SHARED((NP, D), jnp.float32),
            pltpu.SemaphoreType.DMA,
        ],
        name="hetero_attn_pass2",
    )
    return f(v1t, v2t, src1, dst1, src2, dst2, exf, z128)


def _proj_body(fu, fi, wq1, wv1, wk1, wq2, wk2, wv2, bq1, bq2,
               q1o, v1o, k1o, q2o, k2o, v2o):
    u = fu[...]
    it = fi[...]
    q1o[...] = u @ wq1[...] + bq1[...]
    v1o[...] = u @ wv1[...]
    k1o[...] = it @ wk1[...]
    q2o[...] = it @ wq2[...] + bq2[...]
    k2o[...] = it @ wk2[...]
    v2o[...] = it @ wv2[...]


def _proj(fu, fi, Wq1, Wv1, Wk1, Wq2, Wk2, Wv2, bq1, bq2):
    N = fu.shape[0]
    BM = 1000
    grid = (N // BM,)
    row_spec = pl.BlockSpec((BM, D), lambda i: (i, 0))
    w_spec = pl.BlockSpec((D, D), lambda i: (0, 0))
    b_spec = pl.BlockSpec((1, D), lambda i: (0, 0))
    out_sds = jax.ShapeDtypeStruct((N, D), jnp.float32)
    return pl.pallas_call(
        _proj_body,
        grid=grid,
        in_specs=[row_spec, row_spec] + [w_spec] * 6 + [b_spec] * 2,
        out_specs=[row_spec] * 6,
        out_shape=[out_sds] * 6,
    )(fu, fi, Wq1, Wv1, Wk1, Wq2, Wk2, Wv2, bq1.reshape(1, D),
      bq2.reshape(1, D))


def _out_body(a0, a1, d0, d1, rep, fi, wagg, wself, bagg, out):
    den = jnp.maximum(d0[...] + d1[...], 1e-12)
    dr = den @ rep[...]
    agg = (a0[...] + a1[...]) / dr
    out[...] = jnp.maximum(
        agg @ wagg[...] + bagg[...] + fi[...] @ wself[...], 0.0)


def _final(a0, a1, d0, d1, fi, Wagg, Wself, bagg):
    N = fi.shape[0]
    BM = 1000
    grid = (N // BM,)
    row_spec = pl.BlockSpec((BM, D), lambda i: (i, 0))
    den_spec = pl.BlockSpec((BM, H), lambda i: (i, 0))
    rep_spec = pl.BlockSpec((H, D), lambda i: (0, 0))
    w_spec = pl.BlockSpec((D, D), lambda i: (0, 0))
    b_spec = pl.BlockSpec((1, D), lambda i: (0, 0))
    rep = jnp.repeat(jnp.eye(H, dtype=jnp.float32), HD, axis=1)
    return pl.pallas_call(
        _out_body,
        grid=grid,
        in_specs=[row_spec, row_spec, den_spec, den_spec, rep_spec,
                  row_spec, w_spec, w_spec, b_spec],
        out_specs=row_spec,
        out_shape=jax.ShapeDtypeStruct((N, D), jnp.float32),
    )(a0, a1, d0, d1, rep, fi, Wagg, Wself, bagg.reshape(1, D))


def kernel(ft_user, ft_item, edge_index1, cnt1, edge_index2,
           Wq1, bq1, Wk1, Wv1, attn_e1, cnt_table1,
           Wq2, bq2, Wk2, Wv2, attn_e2,
           Wagg, bagg, Wself):
    N = ft_item.shape[0]
    src1 = edge_index1[0]
    dst1 = edge_index1[1]
    src2 = edge_index2[0]
    dst2 = edge_index2[1]

    q1t, v1t, k1t, q2t, k2t, v2t = _proj(
        ft_user, ft_item, Wq1, Wv1, Wk1, Wq2, Wk2, Wv2, bq1, bq2)

    attn = jnp.broadcast_to(
        jnp.concatenate([attn_e1, attn_e2], axis=0)[:, :, None], (2, D, 16))
    z128 = jnp.zeros((NP, D), jnp.float32)

    exf, den2 = _pass1(q1t, k1t, cnt_table1, q2t, k2t, attn,
                       src1, dst1, cnt1, src2, dst2, z128)
    d0 = den2[:NPD].reshape(NPD * 16, H)[:N]
    d1 = den2[NPD:].reshape(NPD * 16, H)[:N]

    agg = _pass2(v1t, v2t, src1, dst1, src2, dst2, exf, z128)
    return _final(agg[:N], agg[NP:NP + N], d0, d1,
                  ft_item, Wagg, Wself, bagg)


# speedup vs baseline: 8.9173x; 1.0537x over previous
import functools

import jax
import jax.numpy as jnp
from jax import lax
from jax.experimental import pallas as pl
from jax.experimental.pallas import tpu as pltpu
from jax.experimental.pallas import tpu_sc as plsc

NC = 2
NS = 16
NW = NC * NS
C = 80
C2 = 64
H = 8
HD = 16
D = 128
NP = 10112
NPD = 640


def _mesh():
    return plsc.VectorSubcoreMesh(core_axis_name="c", subcore_axis_name="s")


def _p1_body(E1, E2, N,
             q1t, k1t, ct1, q2t, k2t, attn, src1, dst1, cnt1, src2, dst2, z128,
             ex_out, den_out,
             sidx0, sidx1, didx0, didx1, cidx0, cidx1,
             qrows0, qrows1, krows0, krows1, ctv, exbuf, denrow, d16, attnv,
             den_scr, semq0, semq1, semk0, semk1):
    c = lax.axis_index("c")
    s = lax.axis_index("s")
    wid = s * NC + c
    RD = NPD // NS
    pltpu.sync_copy(z128.at[pl.ds(s * RD, RD)], den_scr.at[pl.ds(s * RD, RD)])
    pltpu.sync_copy(z128.at[pl.ds(0, C)], denrow)
    pltpu.sync_copy(attn, attnv)
    pltpu.sync_copy(ct1, ctv)
    plsc.subcore_barrier()

    bufs = ((sidx0, didx0, cidx0, qrows0, krows0, semq0, semk0),
            (sidx1, didx1, cidx1, qrows1, krows1, semq1, semk1))

    def make_etype(qt, kt, srcv, dstv, use_cnt, et, ebase, E):
        def fetch(i, p):
            sidx, didx, cidx, qrows, krows, semq, semk = bufs[p]
            base = (wid + i * NW) * C
            pltpu.sync_copy(srcv.at[pl.ds(base, C)], sidx)
            pltpu.sync_copy(dstv.at[pl.ds(base, C)], didx)
            if use_cnt:
                pltpu.sync_copy(cnt1.at[pl.ds(base, C)], cidx)
            pltpu.async_copy(qt.at[sidx], qrows, semq)
            pltpu.async_copy(kt.at[didx], krows, semk)

        def compute(i, p):
            sidx, didx, cidx, qrows, krows, semq, semk = bufs[p]
            base = (wid + i * NW) * C
            pltpu.make_async_copy(qt.at[sidx], qrows, semq).wait()
            pltpu.make_async_copy(kt.at[didx], krows, semk).wait()

            def block_body(b, bc):
                rows = lax.iota(jnp.int32, 16) + b * 16
                dv = plsc.load_gather(didx, [rows])
                plsc.store_scatter(d16, [rows],
                                   lax.shift_right_logical(dv, 4))
                colbase = jnp.bitwise_and(dv, 15) * 8
                if use_cnt:
                    cv = plsc.load_gather(cidx, [rows])
                for h in range(H):
                    acc = jnp.zeros((16,), jnp.float32)
                    for t in range(HD):
                        d = h * HD + t
                        dcol = jnp.full((16,), d, jnp.int32)
                        x = (plsc.load_gather(qrows, [rows, dcol])
                             + plsc.load_gather(krows, [rows, dcol]))
                        if use_cnt:
                            x = x + plsc.load_gather(ctv, [cv, dcol])
                        sg = 1.0 / (1.0 + jnp.exp(-x))
                        acc = acc + attnv[et, d, :] * sg
                    exh = jnp.exp(acc)
                    hcol = jnp.full((16,), h, jnp.int32)
                    plsc.store_scatter(exbuf, [rows, hcol], exh)
                    plsc.store_scatter(denrow, [rows, colbase + h], exh)
                return bc

            lax.fori_loop(0, C // 16, block_body, 0)
            pltpu.sync_copy(exbuf, ex_out.at[pl.ds(ebase + base, C)])
            pltpu.sync_copy(denrow, den_scr.at[d16], add=True)

            def den_clear(b, bc):
                rows = lax.iota(jnp.int32, 16) + b * 16
                dv = plsc.load_gather(didx, [rows])
                colbase = jnp.bitwise_and(dv, 15) * 8
                zv = jnp.zeros((16,), jnp.float32)
                for h in range(H):
                    plsc.store_scatter(denrow, [rows, colbase + h], zv)
                return bc

            lax.fori_loop(0, C // 16, den_clear, 0)

        n = (E // C - 1 - wid) // NW + 1
        fetch(0, 0)

        def pair_body(j, carry):
            i1 = 2 * j + 1

            @pl.when(i1 < n)
            def _():
                fetch(i1, 1)

            compute(2 * j, 0)

            @pl.when(i1 < n)
            def _():
                @pl.when(i1 + 1 < n)
                def _():
                    fetch(i1 + 1, 0)

                compute(i1, 1)

            return carry

        lax.fori_loop(0, (n + 1) // 2, pair_body, 0)

    make_etype(q1t, k1t, src1, dst1, True, 0, 0, E1)
    make_etype(q2t, k2t, src2, dst2, False, 1, E1, E2)
    plsc.subcore_barrier()
    pltpu.sync_copy(den_scr.at[pl.ds(s * RD, RD)],
                    den_out.at[pl.ds(c * NPD + s * RD, RD)])


def _pass1(q1t, k1t, ct1, q2t, k2t, attn, src1, dst1, cnt1, src2, dst2, z128):
    E1 = src1.shape[0]
    E2 = src2.shape[0]
    N = q1t.shape[0]
    NCT = ct1.shape[0]
    f = pl.kernel(
        functools.partial(_p1_body, E1, E2, N),
        out_type=(
            jax.ShapeDtypeStruct((E1 + E2, H), jnp.float32),
            jax.ShapeDtypeStruct((NC * NPD, D), jnp.float32),
        ),
        mesh=_mesh(),
        compiler_params=pltpu.CompilerParams(needs_layout_passes=False),
        scratch_types=[
            pltpu.VMEM((C,), jnp.int32),
            pltpu.VMEM((C,), jnp.int32),
            pltpu.VMEM((C,), jnp.int32),
            pltpu.VMEM((C,), jnp.int32),
            pltpu.VMEM((C,), jnp.int32),
            pltpu.VMEM((C,), jnp.int32),
            pltpu.VMEM((C, D), jnp.float32),
            pltpu.VMEM((C, D), jnp.float32),
            pltpu.VMEM((C, D), jnp.float32),
            pltpu.VMEM((C, D), jnp.float32),
            pltpu.VMEM((NCT, D), jnp.float32),
            pltpu.VMEM((C, H), jnp.float32),
            pltpu.VMEM((C, D), jnp.float32),
            pltpu.VMEM((C,), jnp.int32),
            pltpu.VMEM((2, D, 16), jnp.float32),
            pltpu.VMEM_SHARED((NPD, D), jnp.float32),
            pltpu.SemaphoreType.DMA,
            pltpu.SemaphoreType.DMA,
            pltpu.SemaphoreType.DMA,
            pltpu.SemaphoreType.DMA,
        ],
        name="hetero_attn_pass1",
    )
    return f(q1t, k1t, ct1, q2t, k2t, attn, src1, dst1, cnt1, src2, dst2, z128)


def _p2_body(E1, E2, N,
             v1t, v2t, src1, dst1, src2, dst2, exf, z128,
             agg_out,
             sidx0, sidx1, didx0, didx1, vrows0, vrows1, exb0, exb1, orows,
             agg_scr, semv0, semv1, seme0, seme1):
    c = lax.axis_index("c")
    s = lax.axis_index("s")
    wid = s * NC + c
    R = NP // NS
    pltpu.sync_copy(z128.at[pl.ds(s * R, R)], agg_scr.at[pl.ds(s * R, R)])
    plsc.subcore_barrier()

    bufs = ((sidx0, didx0, vrows0, exb0, semv0, seme0),
            (sidx1, didx1, vrows1, exb1, semv1, seme1))

    def make_etype(vt, srcv, dstv, ebase, E):
        def fetch(i, p):
            sidx, didx, vrows, exb, semv, seme = bufs[p]
            base = (wid + i * NW) * C2
            pltpu.sync_copy(srcv.at[pl.ds(base, C2)], sidx)
            pltpu.sync_copy(dstv.at[pl.ds(base, C2)], didx)
            pltpu.async_copy(vt.at[sidx], vrows, semv)
            pltpu.async_copy(exf.at[pl.ds(ebase + base, C2)], exb, seme)

        def compute(i, p):
            sidx, didx, vrows, exb, semv, seme = bufs[p]
            base = (wid + i * NW) * C2
            pltpu.make_async_copy(vt.at[sidx], vrows, semv).wait()
            pltpu.make_async_copy(
                exf.at[pl.ds(ebase + base, C2)], exb, seme).wait()

            def block_body(b, bc):
                rows = lax.iota(jnp.int32, 16) + b * 16
                for h in range(H):
                    exv = plsc.load_gather(
                        exb, [rows, jnp.full((16,), h, jnp.int32)])
                    for t in range(HD):
                        d = h * HD + t
                        dcol = jnp.full((16,), d, jnp.int32)
                        vv = plsc.load_gather(vrows, [rows, dcol])
                        plsc.store_scatter(orows, [rows, dcol], vv * exv)
                return bc

            lax.fori_loop(0, C2 // 16, block_body, 0)
            pltpu.sync_copy(orows, agg_scr.at[didx], add=True)

        n = (E // C2 - 1 - wid) // NW + 1
        fetch(0, 0)

        def pair_body(j, carry):
            i1 = 2 * j + 1

            @pl.when(i1 < n)
            def _():
                fetch(i1, 1)

            compute(2 * j, 0)

            @pl.when(i1 < n)
            def _():
                @pl.when(i1 + 1 < n)
                def _():
                    fetch(i1 + 1, 0)

                compute(i1, 1)

            return carry

        lax.fori_loop(0, (n + 1) // 2, pair_body, 0)

    make_etype(v1t, src1, dst1, 0, E1)
    make_etype(v2t, src2, dst2, E1, E2)
    plsc.subcore_barrier()
    pltpu.sync_copy(agg_scr.at[pl.ds(s * R, R)],
                    agg_out.at[pl.ds(c * NP + s * R, R)])


def _pass2(v1t, v2t, src1, dst1, src2, dst2, exf, z128):
    E1 = src1.shape[0]
    E2 = src2.shape[0]
    N = v1t.shape[0]
    f = pl.kernel(
        functools.partial(_p2_body, E1, E2, N),
        out_type=jax.ShapeDtypeStruct((NC * NP, D), jnp.float32),
        mesh=_mesh(),
        compiler_params=pltpu.CompilerParams(needs_layout_passes=False),
        scratch_types=[
            pltpu.VMEM((C2,), jnp.int32),
            pltpu.VMEM((C2,), jnp.int32),
            pltpu.VMEM((C2,), jnp.int32),
            pltpu.VMEM((C2,), jnp.int32),
            pltpu.VMEM((C2, D), jnp.float32),
            pltpu.VMEM((C2, D), jnp.float32),
            pltpu.VMEM((C2, H), jnp.float32),
            pltpu.VMEM((C2, H), jnp.float32),
            pltpu.VMEM((C2, D), jnp.float32),
            pltpu.VMEM_SHARED((NP, D), jnp.float32),
            pltpu.SemaphoreType.DMA,
            pltpu.SemaphoreType.DMA,
            pltpu.SemaphoreType.DMA,
            pltpu.SemaphoreType.DMA,
        ],
        name="hetero_attn_pass2",
    )
    return f(v1t, v2t, src1, dst1, src2, dst2, exf, z128)


def _proj_body(fu, fi, wq1, wv1, wk1, wq2, wk2, wv2, bq1, bq2,
               q1o, v1o, k1o, q2o, k2o, v2o):
    u = fu[...]
    it = fi[...]
    q1o[...] = u @ wq1[...] + bq1[...]
    v1o[...] = u @ wv1[...]
    k1o[...] = it @ wk1[...]
    q2o[...] = it @ wq2[...] + bq2[...]
    k2o[...] = it @ wk2[...]
    v2o[...] = it @ wv2[...]


def _proj(fu, fi, Wq1, Wv1, Wk1, Wq2, Wk2, Wv2, bq1, bq2):
    N = fu.shape[0]
    BM = 1000
    grid = (N // BM,)
    row_spec = pl.BlockSpec((BM, D), lambda i: (i, 0))
    w_spec = pl.BlockSpec((D, D), lambda i: (0, 0))
    b_spec = pl.BlockSpec((1, D), lambda i: (0, 0))
    out_sds = jax.ShapeDtypeStruct((N, D), jnp.float32)
    return pl.pallas_call(
        _proj_body,
        grid=grid,
        in_specs=[row_spec, row_spec] + [w_spec] * 6 + [b_spec] * 2,
        out_specs=[row_spec] * 6,
        out_shape=[out_sds] * 6,
    )(fu, fi, Wq1, Wv1, Wk1, Wq2, Wk2, Wv2, bq1.reshape(1, D),
      bq2.reshape(1, D))


def _out_body(a0, a1, d0, d1, rep, fi, wagg, wself, bagg, out):
    den = jnp.maximum(d0[...] + d1[...], 1e-12)
    dr = den @ rep[...]
    agg = (a0[...] + a1[...]) / dr
    out[...] = jnp.maximum(
        agg @ wagg[...] + bagg[...] + fi[...] @ wself[...], 0.0)


def _final(a0, a1, d0, d1, fi, Wagg, Wself, bagg):
    N = fi.shape[0]
    BM = 1000
    grid = (N // BM,)
    row_spec = pl.BlockSpec((BM, D), lambda i: (i, 0))
    den_spec = pl.BlockSpec((BM, H), lambda i: (i, 0))
    rep_spec = pl.BlockSpec((H, D), lambda i: (0, 0))
    w_spec = pl.BlockSpec((D, D), lambda i: (0, 0))
    b_spec = pl.BlockSpec((1, D), lambda i: (0, 0))
    rep = jnp.repeat(jnp.eye(H, dtype=jnp.float32), HD, axis=1)
    return pl.pallas_call(
        _out_body,
        grid=grid,
        in_specs=[row_spec, row_spec, den_spec, den_spec, rep_spec,
                  row_spec, w_spec, w_spec, b_spec],
        out_specs=row_spec,
        out_shape=jax.ShapeDtypeStruct((N, D), jnp.float32),
    )(a0, a1, d0, d1, rep, fi, Wagg, Wself, bagg.reshape(1, D))


def kernel(ft_user, ft_item, edge_index1, cnt1, edge_index2,
           Wq1, bq1, Wk1, Wv1, attn_e1, cnt_table1,
           Wq2, bq2, Wk2, Wv2, attn_e2,
           Wagg, bagg, Wself):
    N = ft_item.shape[0]
    src1 = edge_index1[0]
    dst1 = edge_index1[1]
    src2 = edge_index2[0]
    dst2 = edge_index2[1]

    q1t, v1t, k1t, q2t, k2t, v2t = _proj(
        ft_user, ft_item, Wq1, Wv1, Wk1, Wq2, Wk2, Wv2, bq1, bq2)

    attn = jnp.broadcast_to(
        jnp.concatenate([attn_e1, attn_e2], axis=0)[:, :, None], (2, D, 16))
    z128 = jnp.zeros((NP, D), jnp.float32)

    exf, den2 = _pass1(q1t, k1t, cnt_table1, q2t, k2t, attn,
                       src1, dst1, cnt1, src2, dst2, z128)
    d0 = den2[:NPD].reshape(NPD * 16, H)[:N]
    d1 = den2[NPD:].reshape(NPD * 16, H)[:N]

    agg = _pass2(v1t, v2t, src1, dst1, src2, dst2, exf, z128)
    return _final(agg[:N], agg[NP:NP + N], d0, d1,
                  ft_item, Wagg, Wself, bagg)
